# baseline (speedup 1.0000x reference)
import functools

import jax
import jax.numpy as jnp
from jax import lax
from jax.experimental import pallas as pl
from jax.experimental.pallas import tpu as pltpu
from jax.experimental.pallas import tpu_sc as plsc

B = 2
N = 10000
F = 128
E = 320000
NN = B * N

NC = 2
NT = 16
NW = NC * NT
CH = 128

E_TOT = B * E
EP = 160 * NW * CH
TCH = EP // CH
BLK = 8
RING = 4
DR = TCH // NW
CPT_C0 = 160
CPT_C1 = (TCH - NT * CPT_C0) // NT

DUMMY = NN
ACC_ROWS = 20480
DEG_ROWS = 160

_mesh = plsc.VectorSubcoreMesh(core_axis_name="c", subcore_axis_name="s")


DGB = TCH // NT


def _deg_body(dstd_hbm, zeros_hbm, out_hbm, accv, idxv, redv, racc, part_sh):
    c = lax.axis_index("c")
    s = lax.axis_index("s")

    @pl.when(c == 0)
    def _histogram():
        pltpu.sync_copy(zeros_hbm, accv)
        pltpu.sync_copy(dstd_hbm.at[pl.ds(s * DGB, DGB)], idxv)
        ones = jnp.full((16,), 1.0, jnp.float32)

        def acc_one(i, carry):
            r = i // 8
            j = i - r * 8
            idx = idxv[r, pl.ds(j * 16, 16)]
            plsc.addupdate_scatter(accv, [idx], ones)
            return carry

        lax.fori_loop(0, DGB * 8, acc_one, 0)
        pltpu.sync_copy(accv, part_sh.at[s])

    plsc.subcore_barrier()

    @pl.when(c == 0)
    def _reduce_and_rsqrt():
        base = 1280 * s
        pltpu.sync_copy(part_sh.at[0, pl.ds(base, 1280)], racc)

        def red(p, carry):
            pltpu.sync_copy(part_sh.at[p, pl.ds(base, 1280)], redv)

            def add16(k, carry2):
                sl = pl.ds(k * 16, 16)
                racc[sl] = racc[sl] + redv[sl]
                return carry2

            lax.fori_loop(0, 80, add16, 0)
            return carry

        lax.fori_loop(1, NT, red, 0)

        half = jnp.full((16,), 0.5, jnp.float32)
        th = jnp.full((16,), 1.5, jnp.float32)
        magic = jnp.full((16,), 0x5F3759DF, jnp.int32)
        one = jnp.full((16,), 1.0, jnp.float32)
        c1 = jnp.full((16,), 1, jnp.int32)

        def newton(k, carry):
            sl = pl.ds(k * 16, 16)
            xv = racc[sl] + one
            xi = plsc.bitcast(xv, jnp.int32)
            y = plsc.bitcast(magic - lax.shift_right_logical(xi, c1), jnp.float32)
            hx = xv * half
            y = y * (th - hx * y * y)
            y = y * (th - hx * y * y)
            y = y * (th - hx * y * y)
            racc[sl] = y
            return carry

        lax.fori_loop(0, 80, newton, 0)
        pltpu.sync_copy(racc, out_hbm.at[pl.ds(base, 1280)])


_deg_kernel = functools.partial(
    pl.kernel,
    out_type=jax.ShapeDtypeStruct((DEG_ROWS * 128,), jnp.float32),
    mesh=_mesh,
    scratch_types=[
        pltpu.VMEM((DEG_ROWS * 128,), jnp.float32),
        pltpu.VMEM((DGB, CH), jnp.int32),
        pltpu.VMEM((1280,), jnp.float32),
        pltpu.VMEM((1280,), jnp.float32),
        pltpu.VMEM_SHARED((NT, DEG_ROWS * 128), jnp.float32),
    ],
    compiler_params=pltpu.CompilerParams(needs_layout_passes=False),
)(_deg_body)


def _agg_body(hp_hbm, src_hbm, dst_hbm, out_hbm, srcv, dstv, rows, acc_sh, gsems, ssems):
    c = lax.axis_index("c")
    s = lax.axis_index("s")
    base = jnp.where(c == 0, s * CPT_C0, NT * CPT_C0 + s * CPT_C1)
    cnt = jnp.where(c == 0, CPT_C0, CPT_C1)
    nblk = cnt // BLK
    z = jnp.zeros((32,), jnp.bfloat16)

    def zb(i, carry):
        r = i // 4
        k = i - r * 4
        rows[0, r, pl.ds(k * 32, 32)] = z
        return carry

    lax.fori_loop(0, CH * 4, zb, 0)

    def zacc(j, carry):
        pltpu.sync_copy(rows.at[0], acc_sh.at[pl.ds(1280 * s + CH * j, CH)])
        return carry

    lax.fori_loop(0, 10, zacc, 0)
    pltpu.sync_copy(src_hbm.at[pl.ds(base, BLK)], srcv.at[0])
    pltpu.sync_copy(dst_hbm.at[pl.ds(base, BLK)], dstv.at[0])
    plsc.subcore_barrier()

    for gg in range(RING - 1):
        pltpu.async_copy(hp_hbm.at[srcv.at[0, gg]], rows.at[gg], gsems.at[gg])

    def chunk(g, carry):
        bk = g // BLK
        b = g % RING
        pg = bk % 2
        jg = g - bk * BLK
        pltpu.make_async_copy(
            hp_hbm.at[srcv.at[pg, jg]], rows.at[b], gsems.at[b]
        ).wait()
        pltpu.async_copy(rows.at[b], acc_sh.at[dstv.at[pg, jg]], ssems.at[b], add=True)

        @pl.when(g > 0)
        def _drain_prev():
            g1 = g - 1
            bk1 = g1 // BLK
            b1 = g1 % RING
            pltpu.make_async_copy(
                rows.at[b1],
                acc_sh.at[dstv.at[bk1 % 2, g1 - bk1 * BLK]],
                ssems.at[b1],
            ).wait()

        @pl.when(jnp.logical_and(g == bk * BLK, bk + 1 < nblk))
        def _stage_next():
            p = (bk + 1) % 2
            pltpu.sync_copy(
                src_hbm.at[pl.ds(base + BLK * (bk + 1), BLK)], srcv.at[p]
            )
            pltpu.sync_copy(
                dst_hbm.at[pl.ds(base + BLK * (bk + 1), BLK)], dstv.at[p]
            )

        g2 = g + RING - 1

        @pl.when(g2 < cnt)
        def _issue_next():
            bk2 = g2 // BLK
            pltpu.async_copy(
                hp_hbm.at[srcv.at[bk2 % 2, g2 - bk2 * BLK]],
                rows.at[g2 % RING],
                gsems.at[g2 % RING],
            )

        return carry

    lax.fori_loop(0, cnt, chunk, 0)
    gl = cnt - 1
    glb = gl // BLK
    pltpu.make_async_copy(
        rows.at[gl % RING],
        acc_sh.at[dstv.at[glb % 2, gl - glb * BLK]],
        ssems.at[gl % RING],
    ).wait()
    plsc.subcore_barrier()
    pltpu.sync_copy(
        acc_sh.at[pl.ds(1248 * s, 1248)], out_hbm.at[c, pl.ds(1248 * s, 1248)]
    )

    @pl.when(s == NT - 1)
    def _writeback_tail():
        pltpu.sync_copy(acc_sh.at[pl.ds(19968, 32)], out_hbm.at[c, pl.ds(19968, 32)])


_agg_kernel = functools.partial(
    pl.kernel,
    out_type=jax.ShapeDtypeStruct((NC, NN, F), jnp.bfloat16),
    mesh=_mesh,
    scratch_types=[
        pltpu.VMEM((2, BLK, CH), jnp.int32),
        pltpu.VMEM((2, BLK, CH), jnp.int32),
        pltpu.VMEM((RING, CH, F), jnp.bfloat16),
        pltpu.VMEM_SHARED((ACC_ROWS, F), jnp.bfloat16),
        pltpu.SemaphoreType.DMA((RING,)),
        pltpu.SemaphoreType.DMA((RING,)),
    ],
    compiler_params=pltpu.CompilerParams(
        needs_layout_passes=False, use_tc_tiling_on_sc=False
    ),
)(_agg_body)


BM = 2000
GRID = NN // BM


def _tc1_body(x_ref, w_ref, dinv_ref, hp_ref):
    h = jnp.dot(x_ref[...], w_ref[...], preferred_element_type=jnp.float32)
    hp_ref[...] = (h * dinv_ref[...]).astype(jnp.bfloat16)


def _tc1(xf, W1, dinv):
    return pl.pallas_call(
        _tc1_body,
        grid=(GRID,),
        in_specs=[
            pl.BlockSpec((BM, F), lambda i: (i, 0)),
            pl.BlockSpec((F, F), lambda i: (0, 0)),
            pl.BlockSpec((BM, 1), lambda i: (i, 0)),
        ],
        out_specs=pl.BlockSpec((BM, F), lambda i: (i, 0)),
        out_shape=jax.ShapeDtypeStruct((NN, F), jnp.bfloat16),
    )(xf, W1, dinv)


def _tc2_body(x_ref, hp1_ref, o1_ref, dinv_ref, b1_ref, w2_ref, h_ref, hp2_ref):
    dinv = dinv_ref[...]
    sc = o1_ref[0].astype(jnp.float32) + o1_ref[1].astype(jnp.float32)
    t = sc + hp1_ref[...].astype(jnp.float32)
    conv = dinv * t + b1_ref[...]
    gelu = conv * 0.5 * (1.0 + lax.erf(conv * 0.7071067811865476))
    h = x_ref[...] + gelu
    h_ref[...] = h
    hp2 = jnp.dot(h, w2_ref[...], preferred_element_type=jnp.float32) * dinv
    hp2_ref[...] = hp2.astype(jnp.bfloat16)


def _tc2(xf, hp1, o1, dinv, b1, W2):
    return pl.pallas_call(
        _tc2_body,
        grid=(GRID,),
        in_specs=[
            pl.BlockSpec((BM, F), lambda i: (i, 0)),
            pl.BlockSpec((BM, F), lambda i: (i, 0)),
            pl.BlockSpec((2, BM, F), lambda i: (0, i, 0)),
            pl.BlockSpec((BM, 1), lambda i: (i, 0)),
            pl.BlockSpec((1, F), lambda i: (0, 0)),
            pl.BlockSpec((F, F), lambda i: (0, 0)),
        ],
        out_specs=[
            pl.BlockSpec((BM, F), lambda i: (i, 0)),
            pl.BlockSpec((BM, F), lambda i: (i, 0)),
        ],
        out_shape=[
            jax.ShapeDtypeStruct((NN, F), jnp.float32),
            jax.ShapeDtypeStruct((NN, F), jnp.bfloat16),
        ],
    )(xf, hp1, o1, dinv, b1, W2)


def _tc3_body(h_ref, hp2_ref, o2_ref, dinv_ref, b2_ref, out_ref):
    sc = o2_ref[0].astype(jnp.float32) + o2_ref[1].astype(jnp.float32)
    t = sc + hp2_ref[...].astype(jnp.float32)
    out_ref[...] = h_ref[...] + dinv_ref[...] * t + b2_ref[...]


def _tc3(h, hp2, o2, dinv, b2):
    return pl.pallas_call(
        _tc3_body,
        grid=(GRID,),
        in_specs=[
            pl.BlockSpec((BM, F), lambda i: (i, 0)),
            pl.BlockSpec((BM, F), lambda i: (i, 0)),
            pl.BlockSpec((2, BM, F), lambda i: (0, i, 0)),
            pl.BlockSpec((BM, 1), lambda i: (i, 0)),
            pl.BlockSpec((1, F), lambda i: (0, 0)),
        ],
        out_specs=pl.BlockSpec((BM, F), lambda i: (i, 0)),
        out_shape=jax.ShapeDtypeStruct((NN, F), jnp.float32),
    )(h, hp2, o2, dinv, b2)


def kernel(x, edge_index, W1, b1, W2, b2):
    xf = x.reshape(NN, F)
    offs = (jnp.arange(B, dtype=edge_index.dtype) * N)[:, None, None]
    ei = edge_index + offs
    src = ei[:, 0, :].reshape(-1)
    dst = ei[:, 1, :].reshape(-1)
    pad = EP - E_TOT
    parange = jnp.arange(pad, dtype=jnp.int32)
    src_p = jnp.concatenate([src, parange % NN])
    dst_p = jnp.concatenate([dst, DUMMY + parange % (ACC_ROWS - NN)])
    srcm = src_p.reshape(TCH, CH)
    dstm = dst_p.reshape(TCH, CH)
    zeros = jnp.zeros((DEG_ROWS * 128,), jnp.float32)

    dinv_flat = _deg_kernel(dstm, zeros)
    dinv = dinv_flat.reshape(DEG_ROWS * 128, 1)[:NN]

    hp1 = _tc1(xf, W1, dinv)
    o1 = _agg_kernel(hp1, srcm, dstm)
    h, hp2 = _tc2(xf, hp1, o1, dinv, b1.reshape(1, F), W2)
    o2 = _agg_kernel(hp2, srcm, dstm)
    out = _tc3(h, hp2, o2, dinv, b2.reshape(1, F))
    return out.reshape(B, N, F)


# speedup vs baseline: 1.0801x; 1.0801x over previous
import functools

import jax
import jax.numpy as jnp
from jax import lax
from jax.experimental import pallas as pl
from jax.experimental.pallas import tpu as pltpu
from jax.experimental.pallas import tpu_sc as plsc

B = 2
N = 10000
F = 128
E = 320000
NN = B * N

NC = 2
NT = 16
NW = NC * NT
CH = 128

E_TOT = B * E
EP = 160 * NW * CH
TCH = EP // CH
BLK = 8
RING = 5
DR = TCH // NW
CPT_C0 = 160
CPT_C1 = (TCH - NT * CPT_C0) // NT

DUMMY = NN
ACC_ROWS = 20480
DEG_ROWS = 160

_mesh = plsc.VectorSubcoreMesh(core_axis_name="c", subcore_axis_name="s")


def _deg_body(dstd_hbm, zeros_hbm, out_hbm, accv, idxv):
    c = lax.axis_index("c")
    s = lax.axis_index("s")
    pltpu.sync_copy(zeros_hbm, accv)
    w = s * NC + c
    pltpu.sync_copy(dstd_hbm.at[w], idxv)

    ones = jnp.full((16,), 1.0, jnp.float32)

    def acc_one(i, carry):
        r = i // 8
        j = i - r * 8
        idx = idxv[r, pl.ds(j * 16, 16)]
        plsc.addupdate_scatter(accv, [idx], ones)
        return carry

    lax.fori_loop(0, DR * 8, acc_one, 0)
    pltpu.sync_copy(accv, out_hbm.at[w])


_deg_kernel = functools.partial(
    pl.kernel,
    out_type=jax.ShapeDtypeStruct((NW, DEG_ROWS * 128), jnp.float32),
    mesh=_mesh,
    scratch_types=[
        pltpu.VMEM((DEG_ROWS * 128,), jnp.float32),
        pltpu.VMEM((DR, 128), jnp.int32),
    ],
    compiler_params=pltpu.CompilerParams(needs_layout_passes=False),
)(_deg_body)


def _agg_body(hp_hbm, src_hbm, dst_hbm, out_hbm, srcv, dstv, rows, acc_sh, gsems, ssems):
    c = lax.axis_index("c")
    s = lax.axis_index("s")
    base = jnp.where(c == 0, s * CPT_C0, NT * CPT_C0 + s * CPT_C1)
    cnt = jnp.where(c == 0, CPT_C0, CPT_C1)
    nblk = cnt // BLK
    z = jnp.zeros((32,), jnp.bfloat16)

    def zb(i, carry):
        r = i // 4
        k = i - r * 4
        rows[0, r, pl.ds(k * 32, 32)] = z
        return carry

    lax.fori_loop(0, CH * 4, zb, 0)

    def zacc(j, carry):
        pltpu.sync_copy(rows.at[0], acc_sh.at[pl.ds(1280 * s + CH * j, CH)])
        return carry

    lax.fori_loop(0, 10, zacc, 0)
    pltpu.sync_copy(src_hbm.at[pl.ds(base, BLK)], srcv.at[0])
    pltpu.sync_copy(dst_hbm.at[pl.ds(base, BLK)], dstv.at[0])
    plsc.subcore_barrier()

    for gg in range(RING - 1):
        pltpu.async_copy(hp_hbm.at[srcv.at[0, gg]], rows.at[gg], gsems.at[gg])

    def chunk(g, carry):
        bk = g // BLK
        b = g % RING
        pg = bk % 2
        jg = g - bk * BLK
        pltpu.make_async_copy(
            hp_hbm.at[srcv.at[pg, jg]], rows.at[b], gsems.at[b]
        ).wait()
        pltpu.async_copy(rows.at[b], acc_sh.at[dstv.at[pg, jg]], ssems.at[b], add=True)

        @pl.when(g > 0)
        def _drain_prev():
            g1 = g - 1
            bk1 = g1 // BLK
            b1 = g1 % RING
            pltpu.make_async_copy(
                rows.at[b1],
                acc_sh.at[dstv.at[bk1 % 2, g1 - bk1 * BLK]],
                ssems.at[b1],
            ).wait()

        @pl.when(jnp.logical_and(g == bk * BLK, bk + 1 < nblk))
        def _stage_next():
            p = (bk + 1) % 2
            pltpu.sync_copy(
                src_hbm.at[pl.ds(base + BLK * (bk + 1), BLK)], srcv.at[p]
            )
            pltpu.sync_copy(
                dst_hbm.at[pl.ds(base + BLK * (bk + 1), BLK)], dstv.at[p]
            )

        g2 = g + RING - 1

        @pl.when(g2 < cnt)
        def _issue_next():
            bk2 = g2 // BLK
            pltpu.async_copy(
                hp_hbm.at[srcv.at[bk2 % 2, g2 - bk2 * BLK]],
                rows.at[g2 % RING],
                gsems.at[g2 % RING],
            )

        return carry

    lax.fori_loop(0, cnt, chunk, 0)
    gl = cnt - 1
    glb = gl // BLK
    pltpu.make_async_copy(
        rows.at[gl % RING],
        acc_sh.at[dstv.at[glb % 2, gl - glb * BLK]],
        ssems.at[gl % RING],
    ).wait()
    plsc.subcore_barrier()
    pltpu.sync_copy(
        acc_sh.at[pl.ds(1248 * s, 1248)], out_hbm.at[c, pl.ds(1248 * s, 1248)]
    )

    @pl.when(s == NT - 1)
    def _writeback_tail():
        pltpu.sync_copy(acc_sh.at[pl.ds(19968, 32)], out_hbm.at[c, pl.ds(19968, 32)])


_agg_kernel = functools.partial(
    pl.kernel,
    out_type=jax.ShapeDtypeStruct((NC, NN, F), jnp.bfloat16),
    mesh=_mesh,
    scratch_types=[
        pltpu.VMEM((2, BLK, CH), jnp.int32),
        pltpu.VMEM((2, BLK, CH), jnp.int32),
        pltpu.VMEM((RING, CH, F), jnp.bfloat16),
        pltpu.VMEM_SHARED((ACC_ROWS, F), jnp.bfloat16),
        pltpu.SemaphoreType.DMA((RING,)),
        pltpu.SemaphoreType.DMA((RING,)),
    ],
    compiler_params=pltpu.CompilerParams(
        needs_layout_passes=False, use_tc_tiling_on_sc=False
    ),
)(_agg_body)


BM = 2000
GRID = NN // BM


def _tc0_body(p_ref, r_ref):
    r_ref[...] = lax.rsqrt(1.0 + jnp.sum(p_ref[...], axis=0))


def _tc0(degp):
    return pl.pallas_call(
        _tc0_body,
        out_shape=jax.ShapeDtypeStruct((DEG_ROWS, 128), jnp.float32),
    )(degp)


def _tc1_body(x_ref, w_ref, dinv_ref, hp_ref):
    h = jnp.dot(x_ref[...], w_ref[...], preferred_element_type=jnp.float32)
    hp_ref[...] = (h * dinv_ref[...]).astype(jnp.bfloat16)


def _tc1(xf, W1, dinv):
    return pl.pallas_call(
        _tc1_body,
        grid=(GRID,),
        in_specs=[
            pl.BlockSpec((BM, F), lambda i: (i, 0)),
            pl.BlockSpec((F, F), lambda i: (0, 0)),
            pl.BlockSpec((BM, 1), lambda i: (i, 0)),
        ],
        out_specs=pl.BlockSpec((BM, F), lambda i: (i, 0)),
        out_shape=jax.ShapeDtypeStruct((NN, F), jnp.bfloat16),
    )(xf, W1, dinv)


def _tc2_body(x_ref, hp1_ref, o1_ref, dinv_ref, b1_ref, w2_ref, h_ref, hp2_ref):
    dinv = dinv_ref[...]
    sc = o1_ref[0].astype(jnp.float32) + o1_ref[1].astype(jnp.float32)
    t = sc + hp1_ref[...].astype(jnp.float32)
    conv = dinv * t + b1_ref[...]
    gelu = conv * 0.5 * (1.0 + lax.erf(conv * 0.7071067811865476))
    h = x_ref[...] + gelu
    h_ref[...] = h
    hp2 = jnp.dot(h, w2_ref[...], preferred_element_type=jnp.float32) * dinv
    hp2_ref[...] = hp2.astype(jnp.bfloat16)


def _tc2(xf, hp1, o1, dinv, b1, W2):
    return pl.pallas_call(
        _tc2_body,
        grid=(GRID,),
        in_specs=[
            pl.BlockSpec((BM, F), lambda i: (i, 0)),
            pl.BlockSpec((BM, F), lambda i: (i, 0)),
            pl.BlockSpec((2, BM, F), lambda i: (0, i, 0)),
            pl.BlockSpec((BM, 1), lambda i: (i, 0)),
            pl.BlockSpec((1, F), lambda i: (0, 0)),
            pl.BlockSpec((F, F), lambda i: (0, 0)),
        ],
        out_specs=[
            pl.BlockSpec((BM, F), lambda i: (i, 0)),
            pl.BlockSpec((BM, F), lambda i: (i, 0)),
        ],
        out_shape=[
            jax.ShapeDtypeStruct((NN, F), jnp.float32),
            jax.ShapeDtypeStruct((NN, F), jnp.bfloat16),
        ],
    )(xf, hp1, o1, dinv, b1, W2)


def _tc3_body(h_ref, hp2_ref, o2_ref, dinv_ref, b2_ref, out_ref):
    sc = o2_ref[0].astype(jnp.float32) + o2_ref[1].astype(jnp.float32)
    t = sc + hp2_ref[...].astype(jnp.float32)
    out_ref[...] = h_ref[...] + dinv_ref[...] * t + b2_ref[...]


def _tc3(h, hp2, o2, dinv, b2):
    return pl.pallas_call(
        _tc3_body,
        grid=(GRID,),
        in_specs=[
            pl.BlockSpec((BM, F), lambda i: (i, 0)),
            pl.BlockSpec((BM, F), lambda i: (i, 0)),
            pl.BlockSpec((2, BM, F), lambda i: (0, i, 0)),
            pl.BlockSpec((BM, 1), lambda i: (i, 0)),
            pl.BlockSpec((1, F), lambda i: (0, 0)),
        ],
        out_specs=pl.BlockSpec((BM, F), lambda i: (i, 0)),
        out_shape=jax.ShapeDtypeStruct((NN, F), jnp.float32),
    )(h, hp2, o2, dinv, b2)


def kernel(x, edge_index, W1, b1, W2, b2):
    xf = x.reshape(NN, F)
    offs = (jnp.arange(B, dtype=edge_index.dtype) * N)[:, None, None]
    ei = edge_index + offs
    src = ei[:, 0, :].reshape(-1)
    dst = ei[:, 1, :].reshape(-1)
    pad = EP - E_TOT
    parange = jnp.arange(pad, dtype=jnp.int32)
    src_p = jnp.concatenate([src, parange % NN])
    dst_p = jnp.concatenate([dst, DUMMY + parange % (ACC_ROWS - NN)])
    srcm = src_p.reshape(TCH, CH)
    dstm = dst_p.reshape(TCH, CH)
    zeros = jnp.zeros((DEG_ROWS * 128,), jnp.float32)

    degp = _deg_kernel(dst_p.reshape(NW, DR, CH), zeros)
    r = _tc0(degp.reshape(NW, DEG_ROWS, 128))
    dinv = r.reshape(DEG_ROWS * 128, 1)[:NN]

    hp1 = _tc1(xf, W1, dinv)
    o1 = _agg_kernel(hp1, srcm, dstm)
    h, hp2 = _tc2(xf, hp1, o1, dinv, b1.reshape(1, F), W2)
    o2 = _agg_kernel(hp2, srcm, dstm)
    out = _tc3(h, hp2, o2, dinv, b2.reshape(1, F))
    return out.reshape(B, N, F)


# speedup vs baseline: 1.1527x; 1.0672x over previous
import functools

import jax
import jax.numpy as jnp
from jax import lax
from jax.experimental import pallas as pl
from jax.experimental.pallas import tpu as pltpu
from jax.experimental.pallas import tpu_sc as plsc

B = 2
N = 10000
F = 128
E = 320000
NN = B * N

NC = 2
NT = 16
NW = NC * NT
CH = 128

E_TOT = B * E
EP = 160 * NW * CH
TCH = EP // CH
BLK = 8
RING = 5
DR = TCH // NW
CPT_C0 = 160
CPT_C1 = (TCH - NT * CPT_C0) // NT

DUMMY = NN
ACC_ROWS = 20480
DEG_ROWS = 160

_mesh = plsc.VectorSubcoreMesh(core_axis_name="c", subcore_axis_name="s")


def _deg_body(dstd_hbm, zeros_hbm, out_hbm, accv, idxv):
    c = lax.axis_index("c")
    s = lax.axis_index("s")
    pltpu.sync_copy(zeros_hbm, accv)
    w = s * NC + c
    pltpu.sync_copy(dstd_hbm.at[w], idxv)

    ones = jnp.full((16,), 1.0, jnp.float32)

    def acc_one(i, carry):
        r = i // 8
        j = i - r * 8
        idx = idxv[r, pl.ds(j * 16, 16)]
        plsc.addupdate_scatter(accv, [idx], ones)
        return carry

    lax.fori_loop(0, DR * 8, acc_one, 0)
    pltpu.sync_copy(accv, out_hbm.at[w])


_deg_kernel = functools.partial(
    pl.kernel,
    out_type=jax.ShapeDtypeStruct((NW, DEG_ROWS * 128), jnp.float32),
    mesh=_mesh,
    scratch_types=[
        pltpu.VMEM((DEG_ROWS * 128,), jnp.float32),
        pltpu.VMEM((DR, 128), jnp.int32),
    ],
    compiler_params=pltpu.CompilerParams(needs_layout_passes=False),
)(_deg_body)


JW = BLK - (RING - 1)


def _agg_body(
    hp_hbm, src_hbm, dst_hbm, out_hbm, srcv, dstv, rows, acc_sh, gsems, ssems, stsem
):
    c = lax.axis_index("c")
    s = lax.axis_index("s")
    base = jnp.where(c == 0, s * CPT_C0, NT * CPT_C0 + s * CPT_C1)
    cnt = jnp.where(c == 0, CPT_C0, CPT_C1)
    nblk = cnt // BLK
    z = jnp.zeros((32,), jnp.bfloat16)

    def zb(i, carry):
        r = i // 4
        k = i - r * 4
        rows[0, r, pl.ds(k * 32, 32)] = z
        return carry

    lax.fori_loop(0, CH * 4, zb, 0)

    def zacc(j, carry):
        pltpu.sync_copy(rows.at[0], acc_sh.at[pl.ds(1280 * s + CH * j, CH)])
        return carry

    lax.fori_loop(0, 10, zacc, 0)
    pltpu.sync_copy(src_hbm.at[pl.ds(base, BLK)], srcv.at[0])
    pltpu.sync_copy(dst_hbm.at[pl.ds(base, BLK)], dstv.at[0])
    plsc.subcore_barrier()

    for gg in range(RING - 1):
        pltpu.async_copy(hp_hbm.at[srcv.at[0, gg]], rows.at[gg], gsems.at[gg])

    def chunk(g, carry):
        bk = g // BLK
        b = g % RING
        pg = bk % 2
        jg = g - bk * BLK
        pltpu.make_async_copy(
            hp_hbm.at[srcv.at[pg, jg]], rows.at[b], gsems.at[b]
        ).wait()
        pltpu.async_copy(rows.at[b], acc_sh.at[dstv.at[pg, jg]], ssems.at[b], add=True)

        @pl.when(g > 0)
        def _drain_prev():
            g1 = g - 1
            bk1 = g1 // BLK
            b1 = g1 % RING
            pltpu.make_async_copy(
                rows.at[b1],
                acc_sh.at[dstv.at[bk1 % 2, g1 - bk1 * BLK]],
                ssems.at[b1],
            ).wait()

        @pl.when(jnp.logical_and(jg == 0, bk + 1 < nblk))
        def _stage_next():
            p = (bk + 1) % 2
            pltpu.async_copy(
                src_hbm.at[pl.ds(base + BLK * (bk + 1), BLK)], srcv.at[p], stsem
            )
            pltpu.async_copy(
                dst_hbm.at[pl.ds(base + BLK * (bk + 1), BLK)], dstv.at[p], stsem
            )

        @pl.when(jnp.logical_and(jg == JW, bk + 1 < nblk))
        def _stage_wait():
            p = (bk + 1) % 2
            pltpu.make_async_copy(
                src_hbm.at[pl.ds(base + BLK * (bk + 1), BLK)], srcv.at[p], stsem
            ).wait()
            pltpu.make_async_copy(
                dst_hbm.at[pl.ds(base + BLK * (bk + 1), BLK)], dstv.at[p], stsem
            ).wait()

        g2 = g + RING - 1

        @pl.when(g2 < cnt)
        def _issue_next():
            bk2 = g2 // BLK
            pltpu.async_copy(
                hp_hbm.at[srcv.at[bk2 % 2, g2 - bk2 * BLK]],
                rows.at[g2 % RING],
                gsems.at[g2 % RING],
            )

        return carry

    lax.fori_loop(0, cnt, chunk, 0)
    gl = cnt - 1
    glb = gl // BLK
    pltpu.make_async_copy(
        rows.at[gl % RING],
        acc_sh.at[dstv.at[glb % 2, gl - glb * BLK]],
        ssems.at[gl % RING],
    ).wait()
    plsc.subcore_barrier()
    pltpu.sync_copy(
        acc_sh.at[pl.ds(1248 * s, 1248)], out_hbm.at[c, pl.ds(1248 * s, 1248)]
    )

    @pl.when(s == NT - 1)
    def _writeback_tail():
        pltpu.sync_copy(acc_sh.at[pl.ds(19968, 32)], out_hbm.at[c, pl.ds(19968, 32)])


_agg_kernel = functools.partial(
    pl.kernel,
    out_type=jax.ShapeDtypeStruct((NC, NN, F), jnp.bfloat16),
    mesh=_mesh,
    scratch_types=[
        pltpu.VMEM((2, BLK, CH), jnp.int32),
        pltpu.VMEM((2, BLK, CH), jnp.int32),
        pltpu.VMEM((RING, CH, F), jnp.bfloat16),
        pltpu.VMEM_SHARED((ACC_ROWS, F), jnp.bfloat16),
        pltpu.SemaphoreType.DMA((RING,)),
        pltpu.SemaphoreType.DMA((RING,)),
        pltpu.SemaphoreType.DMA,
    ],
    compiler_params=pltpu.CompilerParams(
        needs_layout_passes=False, use_tc_tiling_on_sc=False
    ),
)(_agg_body)


BM = 2000
GRID = NN // BM


def _tc0_body(p_ref, r_ref):
    r_ref[...] = lax.rsqrt(1.0 + jnp.sum(p_ref[...], axis=0))


def _tc0(degp):
    return pl.pallas_call(
        _tc0_body,
        out_shape=jax.ShapeDtypeStruct((DEG_ROWS, 128), jnp.float32),
    )(degp)


def _tc1_body(x_ref, w_ref, dinv_ref, hp_ref):
    h = jnp.dot(x_ref[...], w_ref[...], preferred_element_type=jnp.float32)
    hp_ref[...] = (h * dinv_ref[...]).astype(jnp.bfloat16)


def _tc1(xf, W1, dinv):
    return pl.pallas_call(
        _tc1_body,
        grid=(GRID,),
        in_specs=[
            pl.BlockSpec((BM, F), lambda i: (i, 0)),
            pl.BlockSpec((F, F), lambda i: (0, 0)),
            pl.BlockSpec((BM, 1), lambda i: (i, 0)),
        ],
        out_specs=pl.BlockSpec((BM, F), lambda i: (i, 0)),
        out_shape=jax.ShapeDtypeStruct((NN, F), jnp.bfloat16),
    )(xf, W1, dinv)


def _tc2_body(x_ref, hp1_ref, o1_ref, dinv_ref, b1_ref, w2_ref, h_ref, hp2_ref):
    dinv = dinv_ref[...]
    sc = o1_ref[0].astype(jnp.float32) + o1_ref[1].astype(jnp.float32)
    t = sc + hp1_ref[...].astype(jnp.float32)
    conv = dinv * t + b1_ref[...]
    gelu = conv * 0.5 * (1.0 + lax.erf(conv * 0.7071067811865476))
    h = x_ref[...] + gelu
    h_ref[...] = h
    hp2 = jnp.dot(h, w2_ref[...], preferred_element_type=jnp.float32) * dinv
    hp2_ref[...] = hp2.astype(jnp.bfloat16)


def _tc2(xf, hp1, o1, dinv, b1, W2):
    return pl.pallas_call(
        _tc2_body,
        grid=(GRID,),
        in_specs=[
            pl.BlockSpec((BM, F), lambda i: (i, 0)),
            pl.BlockSpec((BM, F), lambda i: (i, 0)),
            pl.BlockSpec((2, BM, F), lambda i: (0, i, 0)),
            pl.BlockSpec((BM, 1), lambda i: (i, 0)),
            pl.BlockSpec((1, F), lambda i: (0, 0)),
            pl.BlockSpec((F, F), lambda i: (0, 0)),
        ],
        out_specs=[
            pl.BlockSpec((BM, F), lambda i: (i, 0)),
            pl.BlockSpec((BM, F), lambda i: (i, 0)),
        ],
        out_shape=[
            jax.ShapeDtypeStruct((NN, F), jnp.float32),
            jax.ShapeDtypeStruct((NN, F), jnp.bfloat16),
        ],
    )(xf, hp1, o1, dinv, b1, W2)


def _tc3_body(h_ref, hp2_ref, o2_ref, dinv_ref, b2_ref, out_ref):
    sc = o2_ref[0].astype(jnp.float32) + o2_ref[1].astype(jnp.float32)
    t = sc + hp2_ref[...].astype(jnp.float32)
    out_ref[...] = h_ref[...] + dinv_ref[...] * t + b2_ref[...]


def _tc3(h, hp2, o2, dinv, b2):
    return pl.pallas_call(
        _tc3_body,
        grid=(GRID,),
        in_specs=[
            pl.BlockSpec((BM, F), lambda i: (i, 0)),
            pl.BlockSpec((BM, F), lambda i: (i, 0)),
            pl.BlockSpec((2, BM, F), lambda i: (0, i, 0)),
            pl.BlockSpec((BM, 1), lambda i: (i, 0)),
            pl.BlockSpec((1, F), lambda i: (0, 0)),
        ],
        out_specs=pl.BlockSpec((BM, F), lambda i: (i, 0)),
        out_shape=jax.ShapeDtypeStruct((NN, F), jnp.float32),
    )(h, hp2, o2, dinv, b2)


def kernel(x, edge_index, W1, b1, W2, b2):
    xf = x.reshape(NN, F)
    offs = (jnp.arange(B, dtype=edge_index.dtype) * N)[:, None, None]
    ei = edge_index + offs
    src = ei[:, 0, :].reshape(-1)
    dst = ei[:, 1, :].reshape(-1)
    pad = EP - E_TOT
    parange = jnp.arange(pad, dtype=jnp.int32)
    src_p = jnp.concatenate([src, parange % NN])
    dst_p = jnp.concatenate([dst, DUMMY + parange % (ACC_ROWS - NN)])
    srcm = src_p.reshape(TCH, CH)
    dstm = dst_p.reshape(TCH, CH)
    zeros = jnp.zeros((DEG_ROWS * 128,), jnp.float32)

    degp = _deg_kernel(dst_p.reshape(NW, DR, CH), zeros)
    r = _tc0(degp.reshape(NW, DEG_ROWS, 128))
    dinv = r.reshape(DEG_ROWS * 128, 1)[:NN]

    hp1 = _tc1(xf, W1, dinv)
    o1 = _agg_kernel(hp1, srcm, dstm)
    h, hp2 = _tc2(xf, hp1, o1, dinv, b1.reshape(1, F), W2)
    o2 = _agg_kernel(hp2, srcm, dstm)
    out = _tc3(h, hp2, o2, dinv, b2.reshape(1, F))
    return out.reshape(B, N, F)
